# baseline (speedup 1.0000x reference)
import jax
import jax.numpy as jnp
from jax import lax
from jax.experimental import pallas as pl
from jax.experimental.pallas import tpu as pltpu

_T = 1024


def _kmeans_body(iters_ref, xa_ref, cb0t_ref, out_ref, cbt_ref):
    n = xa_ref.shape[0]
    d = out_ref.shape[1]
    kk = cb0t_ref.shape[1]
    nt = n // _T
    cbt_ref[:] = cb0t_ref[:]

    iota1 = lax.broadcasted_iota(jnp.int32, (_T, kk), 1)

    def outer(_, carry):
        cbt = cbt_ref[:]
        c2 = jnp.sum(cbt * cbt, axis=0, keepdims=True)
        cbt16 = cbt.astype(jnp.bfloat16)

        def tile(j, su):
            xa = xa_ref[pl.ds(j * _T, _T), :]
            x16 = xa[:, :d].astype(jnp.bfloat16)
            g = lax.dot_general(x16, cbt16, (((1,), (0,)), ((), ())),
                                preferred_element_type=jnp.float32)
            dt = c2 - (g + g)
            m = jnp.min(dt, axis=1, keepdims=True)
            idx = jnp.min(jnp.where(dt == m, iota1, kk), axis=1, keepdims=True)
            oh = (iota1 == idx).astype(jnp.bfloat16)
            hi = xa.astype(jnp.bfloat16)
            r1 = xa - hi.astype(jnp.float32)
            mid = r1.astype(jnp.bfloat16)
            lo = (r1 - mid.astype(jnp.float32)).astype(jnp.bfloat16)
            cdims = (((0,), (0,)), ((), ()))
            su = su + lax.dot_general(hi, oh, cdims, preferred_element_type=jnp.float32)
            su = su + lax.dot_general(mid, oh, cdims, preferred_element_type=jnp.float32)
            su = su + lax.dot_general(lo, oh, cdims, preferred_element_type=jnp.float32)
            return su

        su = lax.fori_loop(0, nt, tile, jnp.zeros((d + 1, kk), jnp.float32))
        sums, counts = su[:d, :], su[d:, :]
        mean = sums / jnp.maximum(counts, 1.0)
        cbt_ref[:] = jnp.where(counts > 0.0, mean, cbt)
        return carry

    lax.fori_loop(0, iters_ref[0], outer, 0)
    out_ref[:] = cbt_ref[:].T


def kernel(X, codebook, iters):
    n, d = X.shape
    kk = codebook.shape[0]
    idx = jax.random.permutation(jax.random.key(42), n)[:kk]
    cb0t = X[idx].T
    xa = jnp.concatenate([X, jnp.ones((n, 1), X.dtype)], axis=1)
    it = jnp.asarray(iters, jnp.int32).reshape(1)
    return pl.pallas_call(
        _kmeans_body,
        out_shape=jax.ShapeDtypeStruct((kk, d), X.dtype),
        in_specs=[
            pl.BlockSpec(memory_space=pltpu.SMEM),
            pl.BlockSpec(memory_space=pltpu.VMEM),
            pl.BlockSpec(memory_space=pltpu.VMEM),
        ],
        out_specs=pl.BlockSpec(memory_space=pltpu.VMEM),
        scratch_shapes=[pltpu.VMEM((d, kk), jnp.float32)],
    )(it, xa, cb0t)
